# Initial kernel scaffold; baseline (speedup 1.0000x reference)
import jax
import jax.numpy as jnp
from jax.experimental import pallas as pl


def kernel(atom):
    raise NotImplementedError("write your pallas kernel here")


# speedup vs baseline: 2.8518x; 2.8518x over previous
import functools

import jax
import jax.numpy as jnp
import numpy as np
from jax.experimental import pallas as pl
from jax.experimental.pallas import tpu as pltpu

_EMB_LIST = [100, 11, 11, 11, 9, 4, 9, 5, 4, 8]
_TOTAL = 172
_NFEAT = 10

_OFFSETS = np.concatenate([[0], np.cumsum(_EMB_LIST)[:-1]])

_FEAT_OF_COL = np.repeat(np.arange(_NFEAT), _EMB_LIST)
_LOCAL_OF_COL = np.arange(_TOTAL) - _OFFSETS[_FEAT_OF_COL]

_FEAT_MAP = (np.arange(_NFEAT)[:, None] == _FEAT_OF_COL[None, :]).astype(
    np.float32
)


def _onehot_block(atom_ref, fmap_ref, local_ref, out_ref):
    a = atom_ref[...].astype(jnp.float32)
    g = jax.lax.dot_general(
        a,
        fmap_ref[...],
        (((1,), (0,)), ((), ())),
        preferred_element_type=jnp.float32,
    )
    out_ref[...] = jnp.where(g == local_ref[...], 1.0, 0.0)


def _rows_kernel(atom, block_rows):
    n = atom.shape[0]
    grid = n // block_rows
    fmap = jnp.asarray(_FEAT_MAP)
    local = jnp.asarray(_LOCAL_OF_COL, dtype=jnp.float32)[None, :]
    return pl.pallas_call(
        _onehot_block,
        out_shape=jax.ShapeDtypeStruct((n, _TOTAL), jnp.float32),
        grid=(grid,),
        in_specs=[
            pl.BlockSpec((block_rows, _NFEAT), lambda i: (i, 0)),
            pl.BlockSpec((_NFEAT, _TOTAL), lambda i: (0, 0)),
            pl.BlockSpec((1, _TOTAL), lambda i: (0, 0)),
        ],
        out_specs=pl.BlockSpec((block_rows, _TOTAL), lambda i: (i, 0)),
    )(atom, fmap, local)


@jax.jit
def kernel(atom):
    atom = atom.astype(jnp.int32)
    return _rows_kernel(atom, 2000)


# speedup vs baseline: 3.2004x; 1.1222x over previous
import functools

import jax
import jax.numpy as jnp
import numpy as np
from jax.experimental import pallas as pl
from jax.experimental.pallas import tpu as pltpu

_EMB_LIST = [100, 11, 11, 11, 9, 4, 9, 5, 4, 8]
_TOTAL = 172
_NFEAT = 10

_OFFSETS = np.concatenate([[0], np.cumsum(_EMB_LIST)[:-1]])

_FEAT_OF_COL = np.repeat(np.arange(_NFEAT), _EMB_LIST)
_LOCAL_OF_COL = np.arange(_TOTAL) - _OFFSETS[_FEAT_OF_COL]

_FEAT_MAP = (np.arange(_NFEAT)[:, None] == _FEAT_OF_COL[None, :]).astype(
    np.float32
)


def _onehot_block(atom_ref, fmap_ref, local_ref, out_ref):
    a = atom_ref[...].astype(jnp.float32)
    g = jax.lax.dot_general(
        a,
        fmap_ref[...],
        (((1,), (0,)), ((), ())),
        preferred_element_type=jnp.float32,
    )
    out_ref[...] = jnp.where(g == local_ref[...], 1.0, 0.0)


def _rows_kernel(atom, block_rows):
    n = atom.shape[0]
    grid = n // block_rows
    fmap = jnp.asarray(_FEAT_MAP)
    local = jnp.asarray(_LOCAL_OF_COL, dtype=jnp.float32)[None, :]
    return pl.pallas_call(
        _onehot_block,
        out_shape=jax.ShapeDtypeStruct((n, _TOTAL), jnp.float32),
        grid=(grid,),
        in_specs=[
            pl.BlockSpec((block_rows, _NFEAT), lambda i: (i, 0)),
            pl.BlockSpec((_NFEAT, _TOTAL), lambda i: (0, 0)),
            pl.BlockSpec((1, _TOTAL), lambda i: (0, 0)),
        ],
        out_specs=pl.BlockSpec((block_rows, _TOTAL), lambda i: (i, 0)),
    )(atom, fmap, local)


@jax.jit
def kernel(atom):
    atom = atom.astype(jnp.int32)
    return _rows_kernel(atom, 10000)
